# Initial kernel scaffold; baseline (speedup 1.0000x reference)
import jax
import jax.numpy as jnp
from jax.experimental import pallas as pl


def kernel(user_embed, influencer_embed, pid, edge_index, user_influencer_edges, user_influencer_truth, influencer_influencer_edges, influencer_influencer_truth):
    raise NotImplementedError("write your pallas kernel here")


# speedup vs baseline: 1.9236x; 1.9236x over previous
import functools

import jax
import jax.numpy as jnp
from jax import lax
from jax.experimental import pallas as pl
from jax.experimental.pallas import tpu as pltpu
from jax.experimental.pallas import tpu_sc as plsc

N = 10000
D = 256
E = 160000
NUM_PIDS = 1000
SQRT_EPS = 1e-12
USER_MARGIN = 1.0
INFLUENCER_MARGIN = 1.0
UI_WEIGHT = 1.0
II_WEIGHT = 1.0
NEG_RATIO = 1.0

CH = 128
NCHUNK = 1280
EPAD = NCHUNK * CH
NW = 32
CPW = NCHUNK // NW
NPAD = N + 16


def _sc_body(src1, dst1, ui_s, ui_d, ii_s, ii_d, u_hbm, i_hbm,
             part_sum, part_cnt, dsq_ui_out, dsq_ii_out,
             idx_a, idx_b, rows_a, rows_b, dsq_v, acc_sum, acc_cnt,
             sem_a, sem_b):
    wid = lax.axis_index("s") * 2 + lax.axis_index("c")
    zeros16 = jnp.zeros((16,), jnp.float32)
    ones16 = jnp.ones((16,), jnp.float32)

    def zero_body(i, carry):
        acc_sum[pl.ds(i * 16, 16)] = zeros16
        acc_cnt[pl.ds(i * 16, 16)] = zeros16
        return carry

    lax.fori_loop(0, NPAD // 16, zero_body, 0)

    def dist_chunk(c, src_ref, dst_ref, tab_a, tab_b):
        pltpu.sync_copy(src_ref.at[c], idx_a)
        pltpu.sync_copy(dst_ref.at[c], idx_b)
        cp_a = pltpu.async_copy(tab_a.at[idx_a], rows_a, sem_a)
        cp_b = pltpu.async_copy(tab_b.at[idx_b], rows_b, sem_b)
        cp_a.wait()
        cp_b.wait()

        lane_iota = lax.iota(jnp.int32, 16)

        def group_body(v, carry):
            def edge_body(j, vec):
                e = v * 16 + j
                acc = zeros16
                for k in range(D // 16):
                    du = (rows_a[e, pl.ds(k * 16, 16)]
                          - rows_b[e, pl.ds(k * 16, 16)])
                    acc = acc + du * du
                return jnp.where(lane_iota == j, jnp.sum(acc), vec)

            vec = lax.fori_loop(0, 16, edge_body, zeros16)
            dsq_v[pl.ds(v * 16, 16)] = vec
            return carry

        lax.fori_loop(0, CH // 16, group_body, 0)

    def set1_chunk(j, carry):
        c = wid * CPW + j
        dist_chunk(c, src1, dst1, u_hbm, i_hbm)

        def scat_body(v, inner):
            idxv = idx_b[pl.ds(v * 16, 16)]
            vals = dsq_v[pl.ds(v * 16, 16)]
            plsc.addupdate_scatter(acc_sum, [idxv], vals)
            plsc.addupdate_scatter(acc_cnt, [idxv], ones16)
            return inner

        lax.fori_loop(0, CH // 16, scat_body, 0)
        return carry

    lax.fori_loop(0, CPW, set1_chunk, 0)

    def hinge_loop(src_ref, dst_ref, tab_a, tab_b, out_ref):
        def chunk_body(j, carry):
            c = wid * CPW + j
            dist_chunk(c, src_ref, dst_ref, tab_a, tab_b)
            pltpu.sync_copy(dsq_v, out_ref.at[c])
            return carry

        lax.fori_loop(0, CPW, chunk_body, 0)

    hinge_loop(ui_s, ui_d, u_hbm, i_hbm, dsq_ui_out)
    hinge_loop(ii_s, ii_d, i_hbm, i_hbm, dsq_ii_out)

    pltpu.sync_copy(acc_sum, part_sum.at[wid])
    pltpu.sync_copy(acc_cnt, part_cnt.at[wid])


_sc_fn = pl.kernel(
    _sc_body,
    mesh=plsc.VectorSubcoreMesh(core_axis_name="c", subcore_axis_name="s"),
    compiler_params=pltpu.CompilerParams(needs_layout_passes=False),
    out_type=[
        jax.ShapeDtypeStruct((NW, NPAD), jnp.float32),
        jax.ShapeDtypeStruct((NW, NPAD), jnp.float32),
        jax.ShapeDtypeStruct((NCHUNK, CH), jnp.float32),
        jax.ShapeDtypeStruct((NCHUNK, CH), jnp.float32),
    ],
    scratch_types=[
        pltpu.VMEM((CH,), jnp.int32),
        pltpu.VMEM((CH,), jnp.int32),
        pltpu.VMEM((CH, D), jnp.float32),
        pltpu.VMEM((CH, D), jnp.float32),
        pltpu.VMEM((CH,), jnp.float32),
        pltpu.VMEM((NPAD,), jnp.float32),
        pltpu.VMEM((NPAD,), jnp.float32),
        pltpu.SemaphoreType.DMA,
        pltpu.SemaphoreType.DMA,
    ],
)


def _tc_body(part_sum_ref, part_cnt_ref, pid_ref, dsq_ui_ref, truth_ref,
             dsq_ii_ref, out_ref):
    dist_sum = jnp.sum(part_sum_ref[...], axis=0, keepdims=True)
    cnt = jnp.sum(part_cnt_ref[...], axis=0, keepdims=True)
    follower = jnp.log(dist_sum / jnp.maximum(cnt, 1.0)
                       * (1.0 / (USER_MARGIN * USER_MARGIN)))
    pid = pid_ref[...]

    pos_num = jnp.float32(0.0)
    grp_cnt = jnp.float32(0.0)
    GC = 128
    for g0 in range(0, 1024, GC):
        gi = lax.broadcasted_iota(jnp.int32, (GC, NPAD), 0) + g0
        eq = pid == gi
        gsum = jnp.sum(jnp.where(eq, follower, 0.0), axis=1)
        gcount = jnp.sum(eq.astype(jnp.float32), axis=1)
        present = gcount > 0.0
        gmean = jnp.where(present,
                          jnp.exp(gsum / jnp.maximum(gcount, 1.0)), 0.0)
        pos_num = pos_num + jnp.sum(gmean)
        grp_cnt = grp_cnt + jnp.sum(present.astype(jnp.float32))
    positive = pos_num / grp_cnt

    r_iota = lax.broadcasted_iota(jnp.int32, (NCHUNK, CH), 0)
    c_iota = lax.broadcasted_iota(jnp.int32, (NCHUNK, CH), 1)
    valid = (r_iota * CH + c_iota) < E

    d_ui = jnp.sqrt(dsq_ui_ref[...] + SQRT_EPS)
    neg = (truth_ref[...] == 0) & valid
    terms = jnp.square(jnp.maximum(USER_MARGIN - d_ui, 0.0))
    neg_sum = jnp.sum(jnp.where(neg, terms, 0.0))
    neg_cnt = jnp.sum(neg.astype(jnp.float32))
    negative = neg_sum / jnp.maximum(neg_cnt, 1.0)

    d_ii = jnp.sqrt(dsq_ii_ref[...] + SQRT_EPS)
    ii_terms = jnp.where(
        valid, jnp.square(jnp.maximum(INFLUENCER_MARGIN - d_ii, 0.0)), 0.0)
    ii_mean = jnp.sum(ii_terms) / E

    ui_loss = UI_WEIGHT * (NEG_RATIO * negative + positive)
    total = UI_WEIGHT * ui_loss + II_WEIGHT * (II_WEIGHT * ii_mean)
    out_ref[...] = jnp.broadcast_to(total, (1, 1))


def _pad_edges(edges, pad_dst):
    src = jnp.concatenate(
        [edges[0].astype(jnp.int32), jnp.zeros((EPAD - E,), jnp.int32)])
    dst = jnp.concatenate(
        [edges[1].astype(jnp.int32),
         jnp.full((EPAD - E,), pad_dst, jnp.int32)])
    return src.reshape(NCHUNK, CH), dst.reshape(NCHUNK, CH)


def kernel(user_embed, influencer_embed, pid, edge_index,
           user_influencer_edges, user_influencer_truth,
           influencer_influencer_edges, influencer_influencer_truth):
    src1, dst1 = _pad_edges(edge_index, N)
    ui_s, ui_d = _pad_edges(user_influencer_edges, 0)
    ii_s, ii_d = _pad_edges(influencer_influencer_edges, 0)
    truth = jnp.concatenate(
        [user_influencer_truth.astype(jnp.int32),
         jnp.ones((EPAD - E,), jnp.int32)]).reshape(NCHUNK, CH)
    pid_p = jnp.concatenate(
        [pid.astype(jnp.int32),
         jnp.full((NPAD - N,), 1 << 20, jnp.int32)]).reshape(1, NPAD)

    part_sum, part_cnt, dsq_ui, dsq_ii = _sc_fn(
        src1, dst1, ui_s, ui_d, ii_s, ii_d, user_embed, influencer_embed)

    out = pl.pallas_call(
        _tc_body,
        out_shape=jax.ShapeDtypeStruct((1, 1), jnp.float32),
    )(part_sum, part_cnt, pid_p, dsq_ui, truth, dsq_ii)
    return out.reshape(())


# speedup vs baseline: 2.0284x; 1.0544x over previous
import jax
import jax.numpy as jnp
from jax import lax
from jax.experimental import pallas as pl
from jax.experimental.pallas import tpu as pltpu
from jax.experimental.pallas import tpu_sc as plsc

N = 10000
D = 256
DW = D // 2
E = 160000
NUM_PIDS = 1000
SQRT_EPS = 1e-12
USER_MARGIN = 1.0
INFLUENCER_MARGIN = 1.0
UI_WEIGHT = 1.0
II_WEIGHT = 1.0
NEG_RATIO = 1.0

CH = 128
NCHUNK = 1280
EPAD = NCHUNK * CH
NW = 32
CPW = NCHUNK // NW
NPAD = N + 16


def _sc_body(src1, dst1, ui_s, ui_d, ii_s, ii_d, u_hbm, i_hbm,
             part_sum, part_cnt, dsq_ui_out, dsq_ii_out,
             idx_a0, idx_b0, idx_a1, idx_b1,
             rows_a0, rows_b0, rows_a1, rows_b1,
             dsq_v, acc_sum, acc_cnt,
             sem_a0, sem_b0, sem_a1, sem_b1):
    wid = lax.axis_index("s") * 2 + lax.axis_index("c")
    zeros16 = jnp.zeros((16,), jnp.float32)
    ones16 = jnp.ones((16,), jnp.float32)
    lane_iota = lax.iota(jnp.int32, 16)

    def zero_body(i, carry):
        acc_sum[pl.ds(i * 16, 16)] = zeros16
        acc_cnt[pl.ds(i * 16, 16)] = zeros16
        return carry

    lax.fori_loop(0, NPAD // 16, zero_body, 0)

    def compute_dsq(rows_a, rows_b):
        def group_body(v, carry):
            def edge_body(j, vec):
                e = v * 16 + j
                acc = zeros16
                for k in range(DW // 16):
                    wa = rows_a[e, pl.ds(k * 16, 16)]
                    wb = rows_b[e, pl.ds(k * 16, 16)]
                    a_bf = plsc.bitcast(wa, jnp.bfloat16)
                    b_bf = plsc.bitcast(wb, jnp.bfloat16)
                    du = a_bf - b_bf
                    lo, hi = plsc.unpack(
                        du, format=plsc.PackFormat.INTERLEAVED)
                    acc = acc + lo * lo + hi * hi
                return jnp.where(lane_iota == j, jnp.sum(acc), vec)

            vec = lax.fori_loop(0, 16, edge_body, zeros16)
            dsq_v[pl.ds(v * 16, 16)] = vec
            return carry

        lax.fori_loop(0, CH // 16, group_body, 0)

    def scatter_consume(idx_b):
        def scat_body(v, inner):
            idxv = idx_b[pl.ds(v * 16, 16)]
            vals = dsq_v[pl.ds(v * 16, 16)]
            plsc.addupdate_scatter(acc_sum, [idxv], vals)
            plsc.addupdate_scatter(acc_cnt, [idxv], ones16)
            return inner

        lax.fori_loop(0, CH // 16, scat_body, 0)

    def pair_loop(src_ref, dst_ref, tab_a, tab_b, out_ref):
        def pair_body(t, carry):
            c0 = wid * CPW + 2 * t
            c1 = c0 + 1
            pltpu.sync_copy(src_ref.at[c0], idx_a0)
            pltpu.sync_copy(dst_ref.at[c0], idx_b0)
            pltpu.sync_copy(src_ref.at[c1], idx_a1)
            pltpu.sync_copy(dst_ref.at[c1], idx_b1)
            cp_a0 = pltpu.async_copy(tab_a.at[idx_a0], rows_a0, sem_a0)
            cp_b0 = pltpu.async_copy(tab_b.at[idx_b0], rows_b0, sem_b0)
            cp_a1 = pltpu.async_copy(tab_a.at[idx_a1], rows_a1, sem_a1)
            cp_b1 = pltpu.async_copy(tab_b.at[idx_b1], rows_b1, sem_b1)
            cp_a0.wait()
            cp_b0.wait()
            compute_dsq(rows_a0, rows_b0)
            if out_ref is None:
                scatter_consume(idx_b0)
            else:
                pltpu.sync_copy(dsq_v, out_ref.at[c0])
            cp_a1.wait()
            cp_b1.wait()
            compute_dsq(rows_a1, rows_b1)
            if out_ref is None:
                scatter_consume(idx_b1)
            else:
                pltpu.sync_copy(dsq_v, out_ref.at[c1])
            return carry

        lax.fori_loop(0, CPW // 2, pair_body, 0)

    pair_loop(src1, dst1, u_hbm, i_hbm, None)
    pair_loop(ui_s, ui_d, u_hbm, i_hbm, dsq_ui_out)
    pair_loop(ii_s, ii_d, i_hbm, i_hbm, dsq_ii_out)

    pltpu.sync_copy(acc_sum, part_sum.at[wid])
    pltpu.sync_copy(acc_cnt, part_cnt.at[wid])


_sc_fn = pl.kernel(
    _sc_body,
    mesh=plsc.VectorSubcoreMesh(core_axis_name="c", subcore_axis_name="s"),
    compiler_params=pltpu.CompilerParams(needs_layout_passes=False),
    out_type=[
        jax.ShapeDtypeStruct((NW, NPAD), jnp.float32),
        jax.ShapeDtypeStruct((NW, NPAD), jnp.float32),
        jax.ShapeDtypeStruct((NCHUNK, CH), jnp.float32),
        jax.ShapeDtypeStruct((NCHUNK, CH), jnp.float32),
    ],
    scratch_types=[
        pltpu.VMEM((CH,), jnp.int32),
        pltpu.VMEM((CH,), jnp.int32),
        pltpu.VMEM((CH,), jnp.int32),
        pltpu.VMEM((CH,), jnp.int32),
        pltpu.VMEM((CH, DW), jnp.float32),
        pltpu.VMEM((CH, DW), jnp.float32),
        pltpu.VMEM((CH, DW), jnp.float32),
        pltpu.VMEM((CH, DW), jnp.float32),
        pltpu.VMEM((CH,), jnp.float32),
        pltpu.VMEM((NPAD,), jnp.float32),
        pltpu.VMEM((NPAD,), jnp.float32),
        pltpu.SemaphoreType.DMA,
        pltpu.SemaphoreType.DMA,
        pltpu.SemaphoreType.DMA,
        pltpu.SemaphoreType.DMA,
    ],
)


def _tc_body(part_sum_ref, part_cnt_ref, pid_ref, dsq_ui_ref, truth_ref,
             dsq_ii_ref, out_ref):
    dist_sum = jnp.sum(part_sum_ref[...], axis=0, keepdims=True)
    cnt = jnp.sum(part_cnt_ref[...], axis=0, keepdims=True)
    follower = jnp.log(dist_sum / jnp.maximum(cnt, 1.0)
                       * (1.0 / (USER_MARGIN * USER_MARGIN)))
    pid = pid_ref[...]

    pos_num = jnp.float32(0.0)
    grp_cnt = jnp.float32(0.0)
    GC = 128
    for g0 in range(0, 1024, GC):
        gi = lax.broadcasted_iota(jnp.int32, (GC, NPAD), 0) + g0
        eq = pid == gi
        gsum = jnp.sum(jnp.where(eq, follower, 0.0), axis=1)
        gcount = jnp.sum(eq.astype(jnp.float32), axis=1)
        present = gcount > 0.0
        gmean = jnp.where(present,
                          jnp.exp(gsum / jnp.maximum(gcount, 1.0)), 0.0)
        pos_num = pos_num + jnp.sum(gmean)
        grp_cnt = grp_cnt + jnp.sum(present.astype(jnp.float32))
    positive = pos_num / grp_cnt

    r_iota = lax.broadcasted_iota(jnp.int32, (NCHUNK, CH), 0)
    c_iota = lax.broadcasted_iota(jnp.int32, (NCHUNK, CH), 1)
    valid = (r_iota * CH + c_iota) < E

    d_ui = jnp.sqrt(dsq_ui_ref[...] + SQRT_EPS)
    neg = (truth_ref[...] == 0) & valid
    terms = jnp.square(jnp.maximum(USER_MARGIN - d_ui, 0.0))
    neg_sum = jnp.sum(jnp.where(neg, terms, 0.0))
    neg_cnt = jnp.sum(neg.astype(jnp.float32))
    negative = neg_sum / jnp.maximum(neg_cnt, 1.0)

    d_ii = jnp.sqrt(dsq_ii_ref[...] + SQRT_EPS)
    ii_terms = jnp.where(
        valid, jnp.square(jnp.maximum(INFLUENCER_MARGIN - d_ii, 0.0)), 0.0)
    ii_mean = jnp.sum(ii_terms) / E

    ui_loss = UI_WEIGHT * (NEG_RATIO * negative + positive)
    total = UI_WEIGHT * ui_loss + II_WEIGHT * (II_WEIGHT * ii_mean)
    out_ref[...] = jnp.broadcast_to(total, (1, 1))


def _pad_edges(edges, pad_dst):
    src = jnp.concatenate(
        [edges[0].astype(jnp.int32), jnp.zeros((EPAD - E,), jnp.int32)])
    dst = jnp.concatenate(
        [edges[1].astype(jnp.int32),
         jnp.full((EPAD - E,), pad_dst, jnp.int32)])
    return src.reshape(NCHUNK, CH), dst.reshape(NCHUNK, CH)


def _pack_bf16(table):
    bf = table.astype(jnp.bfloat16).reshape(N, DW, 2)
    return lax.bitcast_convert_type(bf, jnp.float32)


def kernel(user_embed, influencer_embed, pid, edge_index,
           user_influencer_edges, user_influencer_truth,
           influencer_influencer_edges, influencer_influencer_truth):
    src1, dst1 = _pad_edges(edge_index, N)
    ui_s, ui_d = _pad_edges(user_influencer_edges, 0)
    ii_s, ii_d = _pad_edges(influencer_influencer_edges, 0)
    truth = jnp.concatenate(
        [user_influencer_truth.astype(jnp.int32),
         jnp.ones((EPAD - E,), jnp.int32)]).reshape(NCHUNK, CH)
    pid_p = jnp.concatenate(
        [pid.astype(jnp.int32),
         jnp.full((NPAD - N,), 1 << 20, jnp.int32)]).reshape(1, NPAD)

    part_sum, part_cnt, dsq_ui, dsq_ii = _sc_fn(
        src1, dst1, ui_s, ui_d, ii_s, ii_d,
        _pack_bf16(user_embed), _pack_bf16(influencer_embed))

    out = pl.pallas_call(
        _tc_body,
        out_shape=jax.ShapeDtypeStruct((1, 1), jnp.float32),
    )(part_sum, part_cnt, pid_p, dsq_ui, truth, dsq_ii)
    return out.reshape(())
